# baseline (device time: 143109 ns/iter reference)
import jax
import jax.numpy as jnp
from jax import lax
from jax.experimental import pallas as pl
from jax.experimental.pallas import tpu as pltpu

N_DEV = 8
SQ = 256
SKV = 4096
HQ = 8
DH = 128
D = 1024
SCALE = 0.08838834764831843
CHUNK = 512
NCHUNK = SKV // CHUNK


def kernel(x, Wq, Wo, K_ext, V_ext):
    x2 = x.reshape(SQ, D)
    k4 = K_ext.reshape(SKV, HQ * DH)
    v4 = V_ext.reshape(SKV, HQ * DH)

    def body(x_ref, wq_ref, wo_ref, k_ref, v_ref, out_ref,
             qbuf, kbuf, vbuf, ktmp, vtmp, osend, orecv, lsend, lrecv,
             attbuf, q_send_sems, q_recv_sems, o_send_sems, o_recv_sems,
             l_send_sems, l_recv_sems, k_dma_sems, v_dma_sems):
        my = lax.axis_index("i")

        barrier = pltpu.get_barrier_semaphore()
        for t in range(1, N_DEV):
            pl.semaphore_signal(
                barrier, inc=1,
                device_id=((my + t) % N_DEV,),
                device_id_type=pl.DeviceIdType.MESH,
            )

        def start_kv_dma(c):
            slot = c % 2
            ck = pltpu.make_async_copy(
                k_ref.at[pl.ds(c * CHUNK, CHUNK)], ktmp.at[slot],
                k_dma_sems.at[slot])
            cv = pltpu.make_async_copy(
                v_ref.at[pl.ds(c * CHUNK, CHUNK)], vtmp.at[slot],
                v_dma_sems.at[slot])
            ck.start()
            cv.start()
            return ck, cv

        kv_dmas = {0: start_kv_dma(0), 1: start_kv_dma(1)}

        q32 = jnp.dot(x_ref[...].astype(jnp.bfloat16),
                      wq_ref[...].astype(jnp.bfloat16),
                      preferred_element_type=jnp.float32)
        qbuf[0, :, :] = (q32 * SCALE).astype(jnp.bfloat16)

        pl.semaphore_wait(barrier, N_DEV - 1)

        q_sends = []
        for t in range(1, N_DEV):
            rdma = pltpu.make_async_remote_copy(
                src_ref=qbuf.at[0],
                dst_ref=qbuf.at[t],
                send_sem=q_send_sems.at[t],
                recv_sem=q_recv_sems.at[t],
                device_id=((my + t) % N_DEV,),
                device_id_type=pl.DeviceIdType.MESH,
            )
            rdma.start()
            q_sends.append(rdma)

        for c in range(NCHUNK):
            ck, cv = kv_dmas.pop(c)
            ck.wait()
            cv.wait()
            slot = c % 2
            kbuf[c * CHUNK:(c + 1) * CHUNK, :] = ktmp[slot].astype(jnp.bfloat16)
            vbuf[c * CHUNK:(c + 1) * CHUNK, :] = vtmp[slot].astype(jnp.bfloat16)
            if c + 2 < NCHUNK:
                kv_dmas[c + 2] = start_kv_dma(c + 2)

        def compute_block(j, o_dst, l_dst, oslot, lslot):
            for h in range(HQ):
                hs = slice(h * DH, (h + 1) * DH)
                q_h = qbuf[j, :, hs]
                s = lax.dot_general(
                    q_h, kbuf[:, hs],
                    (((1,), (1,)), ((), ())),
                    preferred_element_type=jnp.float32,
                )
                p = jnp.exp(s)
                l_dst[lslot, :, h:h + 1] = jnp.sum(p, axis=1, keepdims=True)
                o_h = lax.dot_general(
                    p.astype(jnp.bfloat16), vbuf[:, hs],
                    (((1,), (0,)), ((), ())),
                    preferred_element_type=jnp.float32,
                )
                o_dst[oslot, :, hs] = o_h.astype(jnp.bfloat16)

        compute_block(0, orecv, lrecv, 0, 0)

        p_sends = []
        for j in range(1, N_DEV):
            q_sends[j - 1].wait_recv()
            slot = N_DEV - j
            compute_block(j, osend, lsend, slot, slot)
            owner = (my - j) % N_DEV
            ro = pltpu.make_async_remote_copy(
                src_ref=osend.at[slot],
                dst_ref=orecv.at[slot],
                send_sem=o_send_sems.at[slot],
                recv_sem=o_recv_sems.at[slot],
                device_id=(owner,),
                device_id_type=pl.DeviceIdType.MESH,
            )
            ro.start()
            rl = pltpu.make_async_remote_copy(
                src_ref=lsend.at[slot],
                dst_ref=lrecv.at[slot],
                send_sem=l_send_sems.at[slot],
                recv_sem=l_recv_sems.at[slot],
                device_id=(owner,),
                device_id_type=pl.DeviceIdType.MESH,
            )
            rl.start()
            p_sends.extend((ro, rl))

        acc_o = orecv[0].astype(jnp.float32)
        acc_l = lrecv[0]
        for j in range(1, N_DEV):
            ro_wait = pltpu.make_async_remote_copy(
                src_ref=osend.at[j], dst_ref=orecv.at[j],
                send_sem=o_send_sems.at[j], recv_sem=o_recv_sems.at[j],
                device_id=(my,), device_id_type=pl.DeviceIdType.MESH,
            )
            ro_wait.wait_recv()
            rl_wait = pltpu.make_async_remote_copy(
                src_ref=lsend.at[j], dst_ref=lrecv.at[j],
                send_sem=l_send_sems.at[j], recv_sem=l_recv_sems.at[j],
                device_id=(my,), device_id_type=pl.DeviceIdType.MESH,
            )
            rl_wait.wait_recv()
            acc_o = acc_o + orecv[j].astype(jnp.float32)
            acc_l = acc_l + lrecv[j]

        for h in range(HQ):
            att_h = acc_o[:, h * DH:(h + 1) * DH] / acc_l[:, h:h + 1]
            attbuf[:, h * DH:(h + 1) * DH] = att_h.astype(jnp.bfloat16)

        out_ref[...] = jnp.dot(attbuf[...], wo_ref[...].astype(jnp.bfloat16),
                               preferred_element_type=jnp.float32)

        for rdma in q_sends + p_sends:
            rdma.wait_send()

    y = pl.pallas_call(
        body,
        out_shape=jax.ShapeDtypeStruct((SQ, D), jnp.float32),
        in_specs=[
            pl.BlockSpec(memory_space=pltpu.VMEM),
            pl.BlockSpec(memory_space=pltpu.VMEM),
            pl.BlockSpec(memory_space=pltpu.VMEM),
            pl.BlockSpec(memory_space=pltpu.HBM),
            pl.BlockSpec(memory_space=pltpu.HBM),
        ],
        out_specs=pl.BlockSpec(memory_space=pltpu.VMEM),
        scratch_shapes=[
            pltpu.VMEM((N_DEV, SQ, D), jnp.bfloat16),
            pltpu.VMEM((SKV, HQ * DH), jnp.bfloat16),
            pltpu.VMEM((SKV, HQ * DH), jnp.bfloat16),
            pltpu.VMEM((2, CHUNK, HQ * DH), jnp.float32),
            pltpu.VMEM((2, CHUNK, HQ * DH), jnp.float32),
            pltpu.VMEM((N_DEV, SQ, D), jnp.bfloat16),
            pltpu.VMEM((N_DEV, SQ, D), jnp.bfloat16),
            pltpu.VMEM((N_DEV, SQ, HQ), jnp.float32),
            pltpu.VMEM((N_DEV, SQ, HQ), jnp.float32),
            pltpu.VMEM((SQ, D), jnp.bfloat16),
            pltpu.SemaphoreType.DMA((N_DEV,)),
            pltpu.SemaphoreType.DMA((N_DEV,)),
            pltpu.SemaphoreType.DMA((N_DEV,)),
            pltpu.SemaphoreType.DMA((N_DEV,)),
            pltpu.SemaphoreType.DMA((N_DEV,)),
            pltpu.SemaphoreType.DMA((N_DEV,)),
            pltpu.SemaphoreType.DMA((2,)),
            pltpu.SemaphoreType.DMA((2,)),
        ],
        compiler_params=pltpu.CompilerParams(
            collective_id=0,
            vmem_limit_bytes=60 * 1024 * 1024,
        ),
    )(x2, Wq, Wo, k4, v4)
    return y.reshape(1, SQ, D)


# device time: 104292 ns/iter; 1.3722x vs baseline; 1.3722x over previous
import jax
import jax.numpy as jnp
from jax import lax
from jax.experimental import pallas as pl
from jax.experimental.pallas import tpu as pltpu

N_DEV = 8
SQ = 256
SKV = 4096
HQ = 8
DH = 128
D = 1024
SCALE = 0.08838834764831843


def kernel(x, Wq, Wo, K_ext, V_ext):
    x2 = x.reshape(SQ, D)
    k4 = K_ext.reshape(SKV, HQ, DH)
    v4 = V_ext.reshape(SKV, HQ, DH)

    def body(x_ref, wq_ref, wo_ref, k_ref, v_ref, out_ref,
             qbuf, kbuf, vbuf, ktmp, vtmp, osend, orecv, lsend, lrecv,
             attbuf, q_send_sems, q_recv_sems, o_send_sems, o_recv_sems,
             l_send_sems, l_recv_sems, k_dma_sems, v_dma_sems):
        my = lax.axis_index("i")

        barrier = pltpu.get_barrier_semaphore()
        for t in range(1, N_DEV):
            pl.semaphore_signal(
                barrier, inc=1,
                device_id=((my + t) % N_DEV,),
                device_id_type=pl.DeviceIdType.MESH,
            )

        def start_kv_dma(h):
            slot = h % 2
            ck = pltpu.make_async_copy(
                k_ref.at[:, h, :], ktmp.at[slot], k_dma_sems.at[slot])
            cv = pltpu.make_async_copy(
                v_ref.at[:, h, :], vtmp.at[slot], v_dma_sems.at[slot])
            ck.start()
            cv.start()
            return ck, cv

        kv_dmas = {0: start_kv_dma(0), 1: start_kv_dma(1)}

        q32 = jnp.dot(x_ref[...].astype(jnp.bfloat16),
                      wq_ref[...].astype(jnp.bfloat16),
                      preferred_element_type=jnp.float32)
        qbuf[0, :, :] = (q32 * SCALE).astype(jnp.bfloat16)

        pl.semaphore_wait(barrier, N_DEV - 1)

        q_sends = []
        for t in range(1, N_DEV):
            rdma = pltpu.make_async_remote_copy(
                src_ref=qbuf.at[0],
                dst_ref=qbuf.at[t],
                send_sem=q_send_sems.at[t],
                recv_sem=q_recv_sems.at[t],
                device_id=((my + t) % N_DEV,),
                device_id_type=pl.DeviceIdType.MESH,
            )
            rdma.start()
            q_sends.append(rdma)

        def attend_head(j, h, o_dst, l_dst, oslot, lslot):
            hs = slice(h * DH, (h + 1) * DH)
            q_h = qbuf[j, :, hs]
            s = lax.dot_general(
                q_h, kbuf[h],
                (((1,), (1,)), ((), ())),
                preferred_element_type=jnp.float32,
            )
            p = jnp.exp(s)
            l_dst[lslot, :, h:h + 1] = jnp.sum(p, axis=1, keepdims=True)
            o_h = lax.dot_general(
                p.astype(jnp.bfloat16), vbuf[h],
                (((1,), (0,)), ((), ())),
                preferred_element_type=jnp.float32,
            )
            o_dst[oslot, :, hs] = o_h.astype(jnp.bfloat16)

        for h in range(HQ):
            ck, cv = kv_dmas.pop(h)
            ck.wait()
            cv.wait()
            slot = h % 2
            kbuf[h, :, :] = ktmp[slot].astype(jnp.bfloat16)
            vbuf[h, :, :] = vtmp[slot].astype(jnp.bfloat16)
            if h + 2 < HQ:
                kv_dmas[h + 2] = start_kv_dma(h + 2)
            attend_head(0, h, orecv, lrecv, 0, 0)

        p_sends = []
        for j in range(1, N_DEV):
            q_sends[j - 1].wait_recv()
            slot = N_DEV - j
            for h in range(HQ):
                attend_head(j, h, osend, lsend, slot, slot)
            owner = (my - j) % N_DEV
            ro = pltpu.make_async_remote_copy(
                src_ref=osend.at[slot],
                dst_ref=orecv.at[slot],
                send_sem=o_send_sems.at[slot],
                recv_sem=o_recv_sems.at[slot],
                device_id=(owner,),
                device_id_type=pl.DeviceIdType.MESH,
            )
            ro.start()
            rl = pltpu.make_async_remote_copy(
                src_ref=lsend.at[slot],
                dst_ref=lrecv.at[slot],
                send_sem=l_send_sems.at[slot],
                recv_sem=l_recv_sems.at[slot],
                device_id=(owner,),
                device_id_type=pl.DeviceIdType.MESH,
            )
            rl.start()
            p_sends.extend((ro, rl))

        acc_o = orecv[0].astype(jnp.float32)
        acc_l = lrecv[0]
        for j in range(1, N_DEV):
            ro_wait = pltpu.make_async_remote_copy(
                src_ref=osend.at[j], dst_ref=orecv.at[j],
                send_sem=o_send_sems.at[j], recv_sem=o_recv_sems.at[j],
                device_id=(my,), device_id_type=pl.DeviceIdType.MESH,
            )
            ro_wait.wait_recv()
            rl_wait = pltpu.make_async_remote_copy(
                src_ref=lsend.at[j], dst_ref=lrecv.at[j],
                send_sem=l_send_sems.at[j], recv_sem=l_recv_sems.at[j],
                device_id=(my,), device_id_type=pl.DeviceIdType.MESH,
            )
            rl_wait.wait_recv()
            acc_o = acc_o + orecv[j].astype(jnp.float32)
            acc_l = acc_l + lrecv[j]

        for h in range(HQ):
            att_h = acc_o[:, h * DH:(h + 1) * DH] / acc_l[:, h:h + 1]
            attbuf[:, h * DH:(h + 1) * DH] = att_h.astype(jnp.bfloat16)

        out_ref[0, :, :] = jnp.dot(attbuf[...], wo_ref[...].astype(jnp.bfloat16),
                                   preferred_element_type=jnp.float32)

        for rdma in q_sends + p_sends:
            rdma.wait_send()

    return pl.pallas_call(
        body,
        out_shape=jax.ShapeDtypeStruct((1, SQ, D), jnp.float32),
        in_specs=[
            pl.BlockSpec(memory_space=pltpu.VMEM),
            pl.BlockSpec(memory_space=pltpu.VMEM),
            pl.BlockSpec(memory_space=pltpu.VMEM),
            pl.BlockSpec(memory_space=pltpu.HBM),
            pl.BlockSpec(memory_space=pltpu.HBM),
        ],
        out_specs=pl.BlockSpec(memory_space=pltpu.VMEM),
        scratch_shapes=[
            pltpu.VMEM((N_DEV, SQ, D), jnp.bfloat16),
            pltpu.VMEM((HQ, SKV, DH), jnp.bfloat16),
            pltpu.VMEM((HQ, SKV, DH), jnp.bfloat16),
            pltpu.VMEM((2, SKV, DH), jnp.float32),
            pltpu.VMEM((2, SKV, DH), jnp.float32),
            pltpu.VMEM((N_DEV, SQ, D), jnp.bfloat16),
            pltpu.VMEM((N_DEV, SQ, D), jnp.bfloat16),
            pltpu.VMEM((N_DEV, SQ, HQ), jnp.float32),
            pltpu.VMEM((N_DEV, SQ, HQ), jnp.float32),
            pltpu.VMEM((SQ, D), jnp.bfloat16),
            pltpu.SemaphoreType.DMA((N_DEV,)),
            pltpu.SemaphoreType.DMA((N_DEV,)),
            pltpu.SemaphoreType.DMA((N_DEV,)),
            pltpu.SemaphoreType.DMA((N_DEV,)),
            pltpu.SemaphoreType.DMA((N_DEV,)),
            pltpu.SemaphoreType.DMA((N_DEV,)),
            pltpu.SemaphoreType.DMA((2,)),
            pltpu.SemaphoreType.DMA((2,)),
        ],
        compiler_params=pltpu.CompilerParams(
            collective_id=0,
            vmem_limit_bytes=62 * 1024 * 1024,
        ),
    )(x2, Wq, Wo, k4, v4)
